# baseline (speedup 1.0000x reference)
import functools

import jax
import jax.numpy as jnp
from jax import lax
from jax.experimental import pallas as pl
from jax.experimental.pallas import tpu as pltpu
from jax.experimental.pallas import tpu_sc as plsc


def kernel(x, table):
    B, S, D = x.shape
    NC, NS = 2, 16
    NW = NC * NS
    SPW = S // NW
    CS = 2
    NCH = SPW // CS
    NITER = NCH // 2
    CHUNK = CS * D
    NBUF = 2 * B

    xf = x.reshape(B * S * D)
    tf = table.reshape(-1)

    mesh = plsc.VectorSubcoreMesh(core_axis_name="c", subcore_axis_name="s")

    @functools.partial(
        pl.kernel,
        out_type=jax.ShapeDtypeStruct((B * S * D,), jnp.float32),
        mesh=mesh,
        scratch_types=[
            pltpu.VMEM((2, CHUNK), jnp.float32),
            pltpu.VMEM((NBUF, CHUNK), jnp.float32),
            pltpu.SemaphoreType.DMA((2,)),
            pltpu.SemaphoreType.DMA((NBUF,)),
            pltpu.SemaphoreType.DMA((NBUF,)),
        ],
    )
    def sc_add(x_hbm, t_hbm, o_hbm, t_v, xo_v, t_sem, in_sem, out_sem):
        wid = lax.axis_index("s") * NC + lax.axis_index("c")
        s_base = wid * SPW

        def t_off(c):
            return (s_base + c * CS) * D

        def x_off(c, b):
            return (b * S + s_base + c * CS) * D

        def fire_t(c, p):
            pltpu.async_copy(
                t_hbm.at[pl.ds(t_off(c), CHUNK)], t_v.at[p], t_sem.at[p]
            )

        def fire_in(c, b, m):
            pltpu.async_copy(
                x_hbm.at[pl.ds(x_off(c, b), CHUNK)], xo_v.at[m], in_sem.at[m]
            )

        for p in range(2):
            fire_t(p, p)
        for p in range(2):
            for b in range(B):
                fire_in(p, b, p * B + b)

        @pl.loop(0, NITER)
        def _iter(cc):
            for p in range(2):
                c = 2 * cc + p
                pltpu.make_async_copy(
                    t_hbm.at[pl.ds(t_off(c), CHUNK)], t_v.at[p], t_sem.at[p]
                ).wait()
                for b in range(B):
                    m = p * B + b
                    pltpu.make_async_copy(
                        x_hbm.at[pl.ds(x_off(c, b), CHUNK)],
                        xo_v.at[m],
                        in_sem.at[m],
                    ).wait()

                    @pl.loop(0, CHUNK // 16, unroll=8)
                    def _vec(i):
                        sl = pl.ds(i * 16, 16)
                        plsc.addupdate(xo_v.at[m, sl], t_v[p, sl])

                    pltpu.async_copy(
                        xo_v.at[m],
                        o_hbm.at[pl.ds(x_off(c, b), CHUNK)],
                        out_sem.at[m],
                    )

                @pl.when(cc < NITER - 1)
                def _():
                    fire_t(2 * (cc + 1) + p, p)

            @pl.when(cc < NITER - 1)
            def _():
                for p in range(2):
                    for b in range(B):
                        m = p * B + b
                        c = 2 * cc + p
                        pltpu.make_async_copy(
                            xo_v.at[m],
                            o_hbm.at[pl.ds(x_off(c, b), CHUNK)],
                            out_sem.at[m],
                        ).wait()
                        fire_in(2 * (cc + 1) + p, b, m)

        for p in range(2):
            for b in range(B):
                m = p * B + b
                c = 2 * (NITER - 1) + p
                pltpu.make_async_copy(
                    xo_v.at[m],
                    o_hbm.at[pl.ds(x_off(c, b), CHUNK)],
                    out_sem.at[m],
                ).wait()

    out = sc_add(xf, tf)
    return out.reshape(B, S, D)


# speedup vs baseline: 4.3010x; 4.3010x over previous
import functools

import jax
import jax.numpy as jnp
from jax import lax
from jax.experimental import pallas as pl
from jax.experimental.pallas import tpu as pltpu
from jax.experimental.pallas import tpu_sc as plsc


def kernel(x, table):
    B, S, D = x.shape
    NC, NS = 2, 16
    NW = NC * NS
    SPW = S // NW
    CS = 4
    NCH = SPW // CS
    NITER = NCH // 2

    mesh = plsc.VectorSubcoreMesh(core_axis_name="c", subcore_axis_name="s")

    @functools.partial(
        pl.kernel,
        out_type=jax.ShapeDtypeStruct((B, S, D), jnp.float32),
        mesh=mesh,
        scratch_types=[
            pltpu.VMEM((2, CS, D), jnp.float32),
            pltpu.VMEM((B, CS, D), jnp.float32),
            pltpu.SemaphoreType.DMA((2,)),
            pltpu.SemaphoreType.DMA((B,)),
            pltpu.SemaphoreType.DMA((B,)),
        ],
    )
    def sc_add(x_hbm, t_hbm, o_hbm, t_v, xo_v, t_sem, in_sem, out_sem):
        wid = lax.axis_index("s") * NC + lax.axis_index("c")
        s_base = wid * SPW

        def s0(c):
            return s_base + c * CS

        def fire_t(c, p):
            pltpu.async_copy(
                t_hbm.at[pl.ds(s0(c), CS), :], t_v.at[p], t_sem.at[p]
            )

        def wait_t(c, p):
            pltpu.make_async_copy(
                t_hbm.at[pl.ds(s0(c), CS), :], t_v.at[p], t_sem.at[p]
            ).wait()

        def fire_in(c, b):
            pltpu.async_copy(
                x_hbm.at[b, pl.ds(s0(c), CS), :], xo_v.at[b], in_sem.at[b]
            )

        def wait_in(c, b):
            pltpu.make_async_copy(
                x_hbm.at[b, pl.ds(s0(c), CS), :], xo_v.at[b], in_sem.at[b]
            ).wait()

        def fire_out(c, b):
            pltpu.async_copy(
                xo_v.at[b], o_hbm.at[b, pl.ds(s0(c), CS), :], out_sem.at[b]
            )

        def wait_out(c, b):
            pltpu.make_async_copy(
                xo_v.at[b], o_hbm.at[b, pl.ds(s0(c), CS), :], out_sem.at[b]
            ).wait()

        fire_t(0, 0)
        fire_t(1, 1)
        for b in range(B):
            fire_in(0, b)

        @pl.loop(0, NITER)
        def _iter(cc):
            for p in range(2):
                c = 2 * cc + p
                wait_t(c, p)
                for b in range(B):
                    wait_in(c, b)
                    for r in range(CS):

                        @pl.loop(0, D // 16, unroll=8)
                        def _vec(i):
                            sl = pl.ds(i * 16, 16)
                            plsc.addupdate(xo_v.at[b, r, sl], t_v[p, r, sl])

                    fire_out(c, b)

                @pl.when(c + 2 < NCH)
                def _():
                    fire_t(c + 2, p)

                @pl.when(c + 1 < NCH)
                def _():
                    for b in range(B):
                        wait_out(c, b)
                        fire_in(c + 1, b)

        for b in range(B):
            wait_out(NCH - 1, b)

    return sc_add(x, table)
